# baseline (speedup 1.0000x reference)
import functools

import jax
import jax.numpy as jnp
from jax import lax
from jax.experimental import pallas as pl
from jax.experimental.pallas import tpu as pltpu
from jax.experimental.pallas import tpu_sc as plsc

D = 1024
B = 4
S = 2048
EPS = 1e-5

NC, NS = 2, 16
NW = NC * NS
LANES = 16
S_PER_W = S // NW
CH = 16
NBLKS = B * S_PER_W // CH
SUBS = S_PER_W // CH
NCHUNK = D // LANES
UNROLL = 8
GROUP = 8

_mesh = plsc.VectorSubcoreMesh(core_axis_name="c", subcore_axis_name="s")


@functools.partial(
    pl.kernel,
    mesh=_mesh,
    compiler_params=pltpu.CompilerParams(needs_layout_passes=False),
    out_type=jax.ShapeDtypeStruct((B * S, D), jnp.float32),
    scratch_types=[
        pltpu.VMEM((NBLKS, CH), jnp.int32),
        pltpu.VMEM((CH, D), jnp.float32),
        pltpu.VMEM((CH, D), jnp.float32),
        pltpu.VMEM((CH, D), jnp.float32),
        pltpu.VMEM((S_PER_W, D), jnp.float32),
        pltpu.VMEM((D,), jnp.float32),
        pltpu.VMEM((D,), jnp.float32),
        pltpu.SemaphoreType.DMA,
        pltpu.SemaphoreType.DMA,
        pltpu.SemaphoreType.DMA,
        pltpu.SemaphoreType.DMA,
        pltpu.SemaphoreType.DMA,
        pltpu.SemaphoreType.DMA,
    ],
)
def _sc_embed(tok_hbm, w_hbm, pe_hbm, scale_hbm, off_hbm, out_hbm,
              idx_v, rows0, rows1, xbuf, pe_v, sc_v, of_v,
              sem_i, sem_pe, sem_g0, sem_g1, sem_w0, sem_w1):
    rows_bufs = (rows0, rows1)
    sem_g = (sem_g0, sem_g1)
    sem_w = (sem_w0, sem_w1)

    wid = lax.axis_index("s") * NC + lax.axis_index("c")
    s0 = wid * S_PER_W

    idx_dma = pltpu.async_copy(tok_hbm.at[wid], idx_v, sem_i)
    pe_dma = pltpu.async_copy(pe_hbm.at[pl.ds(s0, S_PER_W)], pe_v, sem_pe)
    pltpu.sync_copy(scale_hbm, sc_v)
    pltpu.sync_copy(off_hbm, of_v)
    idx_dma.wait()
    pltpu.async_copy(w_hbm.at[idx_v.at[0]], rows_bufs[0], sem_g[0])
    pe_dma.wait()

    def wait_gather(p):
        pltpu.make_async_copy(
            out_hbm.at[pl.ds(0, CH)], rows_bufs[p], sem_g[p]).wait()

    def wait_write(q):
        pltpu.make_async_copy(
            rows_bufs[q], out_hbm.at[pl.ds(0, CH)], sem_w[q]).wait()

    def compute_group(rows_b, pe_base, g):
        rr = [g * GROUP + r for r in range(GROUP)]

        def p1(jj, accs):
            accs = list(accs)
            for u in range(UNROLL):
                off = (jj * UNROLL + u) * LANES
                es = [rows_b[rr[r], pl.ds(off, LANES)] for r in range(GROUP)]
                ps = [pe_v[pe_base + rr[r], pl.ds(off, LANES)]
                      for r in range(GROUP)]
                for r in range(GROUP):
                    x = es[r] + ps[r]
                    xbuf[rr[r], pl.ds(off, LANES)] = x
                    accs[2 * r] = accs[2 * r] + x
                    accs[2 * r + 1] = accs[2 * r + 1] + x * x
            return tuple(accs)

        zero = jnp.zeros((LANES,), jnp.float32)
        accs = lax.fori_loop(0, NCHUNK // UNROLL, p1, (zero,) * (2 * GROUP))

        ys = []
        m2s = []
        for r in range(GROUP):
            mean = jnp.full((LANES,), jnp.sum(accs[2 * r])) * (1.0 / D)
            ex2 = jnp.full((LANES,), jnp.sum(accs[2 * r + 1])) * (1.0 / D)
            vs = ex2 - mean * mean + EPS
            i = lax.bitcast_convert_type(vs, jnp.int32)
            i = jnp.int32(0x5F3759DF) - (i >> 1)
            y = lax.bitcast_convert_type(i, jnp.float32)
            for _ in range(3):
                y = y * (1.5 - 0.5 * vs * y * y)
            ys.append(y)
            m2s.append(mean * y)

        def p2(jj, _):
            for u in range(2):
                j = jj * 2 + u
                scj = sc_v[pl.ds(j * LANES, LANES)]
                ofj = of_v[pl.ds(j * LANES, LANES)]
                xs = [xbuf[rr[r], pl.ds(j * LANES, LANES)]
                      for r in range(GROUP)]
                for r in range(GROUP):
                    t = xs[r] * ys[r] - m2s[r]
                    rows_b[rr[r], pl.ds(j * LANES, LANES)] = t * scj + ofj
            return 0

        lax.fori_loop(0, NCHUNK // 2, p2, 0)

    def do_block(k, p):
        q = 1 - p

        @pl.when(k > 0)
        def _():
            wait_write(q)

        @pl.when(k + 1 < NBLKS)
        def _():
            pltpu.async_copy(
                w_hbm.at[idx_v.at[k + 1]], rows_bufs[q], sem_g[q])

        wait_gather(p)
        b = k // SUBS
        sub = k % SUBS
        for g in range(CH // GROUP):
            compute_group(rows_bufs[p], sub * CH, g)
        t_base = b * S + s0 + sub * CH
        pltpu.async_copy(rows_bufs[p], out_hbm.at[pl.ds(t_base, CH)],
                         sem_w[p])

    def step_body(step, _):
        do_block(step * 2, 0)
        do_block(step * 2 + 1, 1)
        return 0

    lax.fori_loop(0, NBLKS // 2, step_body, 0)
    wait_write(1)


def kernel(token_ids, W_emb, pe, scale, offset):
    tok = (token_ids.astype(jnp.int32)
           .reshape(B, NW, S_PER_W)
           .transpose(1, 0, 2)
           .reshape(NW, NBLKS, CH))
    out = _sc_embed(tok, W_emb, pe, scale, offset)
    return out.reshape(token_ids.shape[0], S, D)


# speedup vs baseline: 1.0709x; 1.0709x over previous
import functools

import jax
import jax.numpy as jnp
from jax import lax
from jax.experimental import pallas as pl
from jax.experimental.pallas import tpu as pltpu
from jax.experimental.pallas import tpu_sc as plsc

D = 1024
B = 4
S = 2048
EPS = 1e-5

NC, NS = 2, 16
NW = NC * NS
LANES = 16
S_PER_W = S // NW
CH = 16
NBLKS = B * S_PER_W // CH
SUBS = S_PER_W // CH
NCHUNK = D // LANES
UNROLL = 8
GROUP = 8

_mesh = plsc.VectorSubcoreMesh(core_axis_name="c", subcore_axis_name="s")


@functools.partial(
    pl.kernel,
    mesh=_mesh,
    compiler_params=pltpu.CompilerParams(needs_layout_passes=False),
    out_type=jax.ShapeDtypeStruct((B * S, D), jnp.float32),
    scratch_types=[
        pltpu.VMEM((NBLKS, CH), jnp.int32),
        pltpu.VMEM((CH, D), jnp.float32),
        pltpu.VMEM((CH, D), jnp.float32),
        pltpu.VMEM((CH, D), jnp.float32),
        pltpu.VMEM((S_PER_W, D), jnp.float32),
        pltpu.VMEM((D,), jnp.float32),
        pltpu.VMEM((D,), jnp.float32),
        pltpu.SemaphoreType.DMA,
        pltpu.SemaphoreType.DMA,
        pltpu.SemaphoreType.DMA,
        pltpu.SemaphoreType.DMA,
        pltpu.SemaphoreType.DMA,
        pltpu.SemaphoreType.DMA,
    ],
)
def _sc_embed(tok_hbm, w_hbm, pe_hbm, scale_hbm, off_hbm, out_hbm,
              idx_v, rows0, rows1, xbuf, pe_v, sc_v, of_v,
              sem_i, sem_pe, sem_g0, sem_g1, sem_w0, sem_w1):
    rows_bufs = (rows0, rows1)
    sem_g = (sem_g0, sem_g1)
    sem_w = (sem_w0, sem_w1)

    wid = lax.axis_index("s") * NC + lax.axis_index("c")
    s0 = wid * S_PER_W

    idx_dma = pltpu.async_copy(tok_hbm.at[wid], idx_v, sem_i)
    pe_dma = pltpu.async_copy(pe_hbm.at[pl.ds(s0, S_PER_W)], pe_v, sem_pe)
    pltpu.sync_copy(scale_hbm, sc_v)
    pltpu.sync_copy(off_hbm, of_v)
    idx_dma.wait()
    pltpu.async_copy(w_hbm.at[idx_v.at[0]], rows_bufs[0], sem_g[0])
    pe_dma.wait()

    def wait_gather(p):
        pltpu.make_async_copy(
            out_hbm.at[pl.ds(0, CH)], rows_bufs[p], sem_g[p]).wait()

    def wait_write(q):
        pltpu.make_async_copy(
            rows_bufs[q], out_hbm.at[pl.ds(0, CH)], sem_w[q]).wait()

    def p1_pass(rows_b, pe_base, g, prev=None):
        rr = [g * GROUP + r for r in range(GROUP)]
        if prev is not None:
            g0, ys0, m2s0 = prev
            rr0 = [g0 * GROUP + r for r in range(GROUP)]

        def body(jj, accs):
            accs = list(accs)
            for u in range(UNROLL):
                off = (jj * UNROLL + u) * LANES
                es = [rows_b[rr[r], pl.ds(off, LANES)] for r in range(GROUP)]
                ps = [pe_v[pe_base + rr[r], pl.ds(off, LANES)]
                      for r in range(GROUP)]
                if prev is not None:
                    scj = sc_v[pl.ds(off, LANES)]
                    ofj = of_v[pl.ds(off, LANES)]
                    xs = [xbuf[rr0[r], pl.ds(off, LANES)]
                          for r in range(GROUP)]
                for r in range(GROUP):
                    x = es[r] + ps[r]
                    xbuf[rr[r], pl.ds(off, LANES)] = x
                    accs[2 * r] = accs[2 * r] + x
                    accs[2 * r + 1] = accs[2 * r + 1] + x * x
                if prev is not None:
                    for r in range(GROUP):
                        t = xs[r] * ys0[r] - m2s0[r]
                        rows_b[rr0[r], pl.ds(off, LANES)] = t * scj + ofj
            return tuple(accs)

        zero = jnp.zeros((LANES,), jnp.float32)
        return lax.fori_loop(0, NCHUNK // UNROLL, body,
                             (zero,) * (2 * GROUP))

    def row_stats(accs):
        ys = []
        m2s = []
        for r in range(GROUP):
            mean = jnp.full((LANES,), jnp.sum(accs[2 * r])) * (1.0 / D)
            ex2 = jnp.full((LANES,), jnp.sum(accs[2 * r + 1])) * (1.0 / D)
            vs = ex2 - mean * mean + EPS
            i = lax.bitcast_convert_type(vs, jnp.int32)
            i = jnp.int32(0x5F3759DF) - (i >> 1)
            y = lax.bitcast_convert_type(i, jnp.float32)
            for _ in range(3):
                y = y * (1.5 - 0.5 * vs * y * y)
            ys.append(y)
            m2s.append(mean * y)
        return ys, m2s

    def p2_pass(rows_b, g, ys, m2s):
        rr = [g * GROUP + r for r in range(GROUP)]

        def body(jj, _):
            for u in range(2):
                j = jj * 2 + u
                scj = sc_v[pl.ds(j * LANES, LANES)]
                ofj = of_v[pl.ds(j * LANES, LANES)]
                xs = [xbuf[rr[r], pl.ds(j * LANES, LANES)]
                      for r in range(GROUP)]
                for r in range(GROUP):
                    t = xs[r] * ys[r] - m2s[r]
                    rows_b[rr[r], pl.ds(j * LANES, LANES)] = t * scj + ofj
            return 0

        lax.fori_loop(0, NCHUNK // 2, body, 0)

    def do_block(k, p):
        q = 1 - p

        @pl.when(k > 0)
        def _():
            wait_write(q)

        @pl.when(k + 1 < NBLKS)
        def _():
            pltpu.async_copy(
                w_hbm.at[idx_v.at[k + 1]], rows_bufs[q], sem_g[q])

        wait_gather(p)
        b = k // SUBS
        sub = k % SUBS
        rows_b = rows_bufs[p]
        accs0 = p1_pass(rows_b, sub * CH, 0)
        ys0, m2s0 = row_stats(accs0)
        accs1 = p1_pass(rows_b, sub * CH, 1, prev=(0, ys0, m2s0))
        ys1, m2s1 = row_stats(accs1)
        p2_pass(rows_b, 1, ys1, m2s1)
        t_base = b * S + s0 + sub * CH
        pltpu.async_copy(rows_bufs[p], out_hbm.at[pl.ds(t_base, CH)],
                         sem_w[p])

    def step_body(step, _):
        do_block(step * 2, 0)
        do_block(step * 2 + 1, 1)
        return 0

    lax.fori_loop(0, NBLKS // 2, step_body, 0)
    wait_write(1)


def kernel(token_ids, W_emb, pe, scale, offset):
    tok = (token_ids.astype(jnp.int32)
           .reshape(B, NW, S_PER_W)
           .transpose(1, 0, 2)
           .reshape(NW, NBLKS, CH))
    out = _sc_embed(tok, W_emb, pe, scale, offset)
    return out.reshape(token_ids.shape[0], S, D)


# speedup vs baseline: 1.1993x; 1.1198x over previous
import functools

import jax
import jax.numpy as jnp
from jax import lax
from jax.experimental import pallas as pl
from jax.experimental.pallas import tpu as pltpu
from jax.experimental.pallas import tpu_sc as plsc

D = 1024
B = 4
S = 2048
EPS = 1e-5

NC, NS = 2, 16
NW = NC * NS
LANES = 16
S_PER_W = S // NW
CH = 16
NBLKS = B * S_PER_W // CH
SUBS = S_PER_W // CH
NCHUNK = D // LANES
UNROLL = 8
GROUP = 8

_mesh = plsc.VectorSubcoreMesh(core_axis_name="c", subcore_axis_name="s")


@functools.partial(
    pl.kernel,
    mesh=_mesh,
    compiler_params=pltpu.CompilerParams(needs_layout_passes=False),
    out_type=jax.ShapeDtypeStruct((B * S, D), jnp.float32),
    scratch_types=[
        pltpu.VMEM((NBLKS, CH), jnp.int32),
        pltpu.VMEM((CH, D), jnp.float32),
        pltpu.VMEM((CH, D), jnp.float32),
        pltpu.VMEM((CH, D), jnp.float32),
        pltpu.VMEM((S_PER_W, D), jnp.float32),
        pltpu.VMEM((D,), jnp.float32),
        pltpu.VMEM((D,), jnp.float32),
        pltpu.SemaphoreType.DMA,
        pltpu.SemaphoreType.DMA,
        pltpu.SemaphoreType.DMA,
        pltpu.SemaphoreType.DMA,
        pltpu.SemaphoreType.DMA,
        pltpu.SemaphoreType.DMA,
    ],
)
def _sc_embed(tok_hbm, w_hbm, pe_hbm, scale_hbm, off_hbm, out_hbm,
              idx_v, rows0, rows1, xbuf, pe_v, sc_v, of_v,
              sem_i, sem_pe, sem_g0, sem_g1, sem_w0, sem_w1):
    rows_bufs = (rows0, rows1)
    sem_g = (sem_g0, sem_g1)
    sem_w = (sem_w0, sem_w1)

    wid = lax.axis_index("s") * NC + lax.axis_index("c")
    s0 = wid * S_PER_W

    idx_dma = pltpu.async_copy(tok_hbm.at[wid], idx_v, sem_i)
    pe_dma = pltpu.async_copy(pe_hbm.at[pl.ds(s0, S_PER_W)], pe_v, sem_pe)
    pltpu.sync_copy(scale_hbm, sc_v)
    pltpu.sync_copy(off_hbm, of_v)
    idx_dma.wait()
    pltpu.async_copy(w_hbm.at[idx_v.at[0]], rows_bufs[0], sem_g[0])
    pe_dma.wait()

    def wait_gather(p):
        pltpu.make_async_copy(
            out_hbm.at[pl.ds(0, CH)], rows_bufs[p], sem_g[p]).wait()

    def wait_write(q):
        pltpu.make_async_copy(
            rows_bufs[q], out_hbm.at[pl.ds(0, CH)], sem_w[q]).wait()

    def p1_pass(rows_b, pe_base, g, prev=None):
        rr = [g * GROUP + r for r in range(GROUP)]
        if prev is not None:
            g0, ys0, m2s0 = prev
            rr0 = [g0 * GROUP + r for r in range(GROUP)]

        def body(jj, accs):
            accs = list(accs)
            for u in range(UNROLL):
                off = (jj * UNROLL + u) * LANES
                es = [rows_b[rr[r], pl.ds(off, LANES)] for r in range(GROUP)]
                ps = [pe_v[pe_base + rr[r], pl.ds(off, LANES)]
                      for r in range(GROUP)]
                if prev is not None:
                    scj = sc_v[pl.ds(off, LANES)]
                    ofj = of_v[pl.ds(off, LANES)]
                    xs = [xbuf[rr0[r], pl.ds(off, LANES)]
                          for r in range(GROUP)]
                for r in range(GROUP):
                    x = es[r] + ps[r]
                    xbuf[rr[r], pl.ds(off, LANES)] = x
                    accs[2 * r] = accs[2 * r] + x
                    accs[2 * r + 1] = accs[2 * r + 1] + x * x
                if prev is not None:
                    for r in range(GROUP):
                        t = xs[r] * ys0[r] - m2s0[r]
                        rows_b[rr0[r], pl.ds(off, LANES)] = t * scj + ofj
            return tuple(accs)

        zero = jnp.zeros((LANES,), jnp.float32)
        return plsc.parallel_loop(
            0, NCHUNK // UNROLL, carry=(zero,) * (2 * GROUP))(body)

    def row_stats(accs):
        ys = []
        m2s = []
        for r in range(GROUP):
            mean = jnp.full((LANES,), jnp.sum(accs[2 * r])) * (1.0 / D)
            ex2 = jnp.full((LANES,), jnp.sum(accs[2 * r + 1])) * (1.0 / D)
            vs = ex2 - mean * mean + EPS
            i = lax.bitcast_convert_type(vs, jnp.int32)
            i = jnp.int32(0x5F3759DF) - (i >> 1)
            y = lax.bitcast_convert_type(i, jnp.float32)
            for _ in range(3):
                y = y * (1.5 - 0.5 * vs * y * y)
            ys.append(y)
            m2s.append(mean * y)
        return ys, m2s

    def p2_pass(rows_b, g, ys, m2s):
        rr = [g * GROUP + r for r in range(GROUP)]

        def body(jj):
            for u in range(2):
                j = jj * 2 + u
                scj = sc_v[pl.ds(j * LANES, LANES)]
                ofj = of_v[pl.ds(j * LANES, LANES)]
                xs = [xbuf[rr[r], pl.ds(j * LANES, LANES)]
                      for r in range(GROUP)]
                for r in range(GROUP):
                    t = xs[r] * ys[r] - m2s[r]
                    rows_b[rr[r], pl.ds(j * LANES, LANES)] = t * scj + ofj

        plsc.parallel_loop(0, NCHUNK // 2)(body)

    def do_block(k, p):
        q = 1 - p

        @pl.when(k > 0)
        def _():
            wait_write(q)

        @pl.when(k + 1 < NBLKS)
        def _():
            pltpu.async_copy(
                w_hbm.at[idx_v.at[k + 1]], rows_bufs[q], sem_g[q])

        wait_gather(p)
        b = k // SUBS
        sub = k % SUBS
        rows_b = rows_bufs[p]
        accs0 = p1_pass(rows_b, sub * CH, 0)
        ys0, m2s0 = row_stats(accs0)
        accs1 = p1_pass(rows_b, sub * CH, 1)
        ys1, m2s1 = row_stats(accs1)
        p2_pass(rows_b, 0, ys0, m2s0)
        p2_pass(rows_b, 1, ys1, m2s1)
        t_base = b * S + s0 + sub * CH
        pltpu.async_copy(rows_bufs[p], out_hbm.at[pl.ds(t_base, CH)],
                         sem_w[p])

    def step_body(step, _):
        do_block(step * 2, 0)
        do_block(step * 2 + 1, 1)
        return 0

    lax.fori_loop(0, NBLKS // 2, step_body, 0)
    wait_write(1)


def kernel(token_ids, W_emb, pe, scale, offset):
    tok = (token_ids.astype(jnp.int32)
           .reshape(B, NW, S_PER_W)
           .transpose(1, 0, 2)
           .reshape(NW, NBLKS, CH))
    out = _sc_embed(tok, W_emb, pe, scale, offset)
    return out.reshape(token_ids.shape[0], S, D)


# speedup vs baseline: 1.2103x; 1.0092x over previous
import functools

import jax
import jax.numpy as jnp
from jax import lax
from jax.experimental import pallas as pl
from jax.experimental.pallas import tpu as pltpu
from jax.experimental.pallas import tpu_sc as plsc

D = 1024
B = 4
S = 2048
EPS = 1e-5

NC, NS = 2, 16
NW = NC * NS
LANES = 16
S_PER_W = S // NW
CH = 16
NBLKS = B * S_PER_W // CH
SUBS = S_PER_W // CH
NCHUNK = D // LANES
UNROLL = 8
GROUP = 8

_mesh = plsc.VectorSubcoreMesh(core_axis_name="c", subcore_axis_name="s")


@functools.partial(
    pl.kernel,
    mesh=_mesh,
    compiler_params=pltpu.CompilerParams(needs_layout_passes=False),
    out_type=jax.ShapeDtypeStruct((B * S, D), jnp.float32),
    scratch_types=[
        pltpu.VMEM((NBLKS, CH), jnp.int32),
        pltpu.VMEM((CH, D), jnp.float32),
        pltpu.VMEM((CH, D), jnp.float32),
        pltpu.VMEM((CH, D), jnp.float32),
        pltpu.VMEM((S_PER_W, D), jnp.float32),
        pltpu.VMEM((D,), jnp.float32),
        pltpu.VMEM((D,), jnp.float32),
        pltpu.SemaphoreType.DMA,
        pltpu.SemaphoreType.DMA,
        pltpu.SemaphoreType.DMA,
        pltpu.SemaphoreType.DMA,
        pltpu.SemaphoreType.DMA,
        pltpu.SemaphoreType.DMA,
    ],
)
def _sc_embed(tok_hbm, w_hbm, pe_hbm, scale_hbm, off_hbm, out_hbm,
              idx_v, rows0, rows1, xbuf, pe_v, sc_v, of_v,
              sem_i, sem_pe, sem_g0, sem_g1, sem_w0, sem_w1):
    rows_bufs = (rows0, rows1)
    sem_g = (sem_g0, sem_g1)
    sem_w = (sem_w0, sem_w1)

    wid = lax.axis_index("s") * NC + lax.axis_index("c")
    s0 = wid * S_PER_W

    idx_dma = pltpu.async_copy(tok_hbm.at[wid], idx_v, sem_i)
    pe_dma = pltpu.async_copy(pe_hbm.at[pl.ds(s0, S_PER_W)], pe_v, sem_pe)
    pltpu.sync_copy(scale_hbm, sc_v)
    pltpu.sync_copy(off_hbm, of_v)
    idx_dma.wait()
    pltpu.async_copy(w_hbm.at[idx_v.at[0]], rows_bufs[0], sem_g[0])
    pe_dma.wait()

    def wait_gather(p):
        pltpu.make_async_copy(
            out_hbm.at[pl.ds(0, CH)], rows_bufs[p], sem_g[p]).wait()

    def wait_write(q):
        pltpu.make_async_copy(
            rows_bufs[q], out_hbm.at[pl.ds(0, CH)], sem_w[q]).wait()

    def p1_pass(rows_b, pe_base, g, prev=None):
        rr = [g * GROUP + r for r in range(GROUP)]
        if prev is not None:
            g0, ys0, m2s0 = prev
            rr0 = [g0 * GROUP + r for r in range(GROUP)]

        def body(jj, accs):
            accs = list(accs)
            for u in range(UNROLL):
                off = (jj * UNROLL + u) * LANES
                es = [rows_b[rr[r], pl.ds(off, LANES)] for r in range(GROUP)]
                ps = [pe_v[pe_base + rr[r], pl.ds(off, LANES)]
                      for r in range(GROUP)]
                if prev is not None:
                    scj = sc_v[pl.ds(off, LANES)]
                    ofj = of_v[pl.ds(off, LANES)]
                    xs = [xbuf[rr0[r], pl.ds(off, LANES)]
                          for r in range(GROUP)]
                for r in range(GROUP):
                    x = es[r] + ps[r]
                    xbuf[rr[r], pl.ds(off, LANES)] = x
                    accs[2 * r] = accs[2 * r] + x
                    accs[2 * r + 1] = accs[2 * r + 1] + x * x
                if prev is not None:
                    for r in range(GROUP):
                        t = xs[r] * ys0[r] - m2s0[r]
                        rows_b[rr0[r], pl.ds(off, LANES)] = t * scj + ofj
            return tuple(accs)

        zero = jnp.zeros((LANES,), jnp.float32)
        return plsc.parallel_loop(
            0, NCHUNK // UNROLL, carry=(zero,) * (2 * GROUP))(body)

    def row_stats(accs):
        ys = []
        m2s = []
        for r in range(GROUP):
            mean = jnp.full((LANES,), jnp.sum(accs[2 * r])) * (1.0 / D)
            ex2 = jnp.full((LANES,), jnp.sum(accs[2 * r + 1])) * (1.0 / D)
            vs = ex2 - mean * mean + EPS
            i = lax.bitcast_convert_type(vs, jnp.int32)
            i = jnp.int32(0x5F3759DF) - (i >> 1)
            y = lax.bitcast_convert_type(i, jnp.float32)
            for _ in range(3):
                y = y * (1.5 - 0.5 * vs * y * y)
            ys.append(y)
            m2s.append(mean * y)
        return ys, m2s

    def p2_pass(rows_b, g, ys, m2s):
        rr = [g * GROUP + r for r in range(GROUP)]

        def body(j):
            scj = sc_v[pl.ds(j * LANES, LANES)]
            ofj = of_v[pl.ds(j * LANES, LANES)]
            xs = [xbuf[rr[r], pl.ds(j * LANES, LANES)]
                  for r in range(GROUP)]
            for r in range(GROUP):
                t = xs[r] * ys[r] - m2s[r]
                rows_b[rr[r], pl.ds(j * LANES, LANES)] = t * scj + ofj

        plsc.parallel_loop(0, NCHUNK, unroll=2)(body)

    def do_block(k, p):
        q = 1 - p

        @pl.when(k > 0)
        def _():
            wait_write(q)

        @pl.when(k + 1 < NBLKS)
        def _():
            pltpu.async_copy(
                w_hbm.at[idx_v.at[k + 1]], rows_bufs[q], sem_g[q])

        wait_gather(p)
        b = k // SUBS
        sub = k % SUBS
        rows_b = rows_bufs[p]
        accs0 = p1_pass(rows_b, sub * CH, 0)
        ys0, m2s0 = row_stats(accs0)
        accs1 = p1_pass(rows_b, sub * CH, 1)
        ys1, m2s1 = row_stats(accs1)
        p2_pass(rows_b, 0, ys0, m2s0)
        p2_pass(rows_b, 1, ys1, m2s1)
        t_base = b * S + s0 + sub * CH
        pltpu.async_copy(rows_bufs[p], out_hbm.at[pl.ds(t_base, CH)],
                         sem_w[p])

    def step_body(step, _):
        do_block(step * 2, 0)
        do_block(step * 2 + 1, 1)
        return 0

    lax.fori_loop(0, NBLKS // 2, step_body, 0)
    wait_write(1)


def kernel(token_ids, W_emb, pe, scale, offset):
    tok = (token_ids.astype(jnp.int32)
           .reshape(B, NW, S_PER_W)
           .transpose(1, 0, 2)
           .reshape(NW, NBLKS, CH))
    out = _sc_embed(tok, W_emb, pe, scale, offset)
    return out.reshape(token_ids.shape[0], S, D)
